# Initial kernel scaffold; baseline (speedup 1.0000x reference)
import jax
import jax.numpy as jnp
from jax.experimental import pallas as pl


def kernel(x, table):
    raise NotImplementedError("write your pallas kernel here")


# speedup vs baseline: 1.1019x; 1.1019x over previous
import functools

import jax
import jax.numpy as jnp
from jax import lax
from jax.experimental import pallas as pl
from jax.experimental.pallas import tpu as pltpu
from jax.experimental.pallas import tpu_sc as plsc

BATCH = 16384
HIST = 50
EMBED_DIM = 32
TOTAL = BATCH * HIST
NUM_CORES = 2
NUM_SUBCORES = 16
NW = NUM_CORES * NUM_SUBCORES
B_PER_W = TOTAL // NW
CHUNK = 1600
NCHUNK = B_PER_W // CHUNK

_mesh = plsc.VectorSubcoreMesh(core_axis_name="c", subcore_axis_name="s")


@functools.partial(
    pl.kernel,
    mesh=_mesh,
    out_type=jax.ShapeDtypeStruct((TOTAL, EMBED_DIM), jnp.float32),
    scratch_types=[
        pltpu.VMEM((CHUNK,), jnp.int32),
        pltpu.VMEM((CHUNK, EMBED_DIM), jnp.float32),
        pltpu.SemaphoreType.DMA,
    ],
    compiler_params=pltpu.CompilerParams(use_tc_tiling_on_sc=False),
)
def _emb_lookup(x_hbm, table_hbm, out_hbm, idx_v, rows_v, sem):
    wid = lax.axis_index("s") * NUM_CORES + lax.axis_index("c")
    base = wid * B_PER_W

    def body(i, _):
        off = base + i * CHUNK
        pltpu.sync_copy(x_hbm.at[pl.ds(off, CHUNK)], idx_v)
        pltpu.async_copy(table_hbm.at[idx_v], rows_v, sem).wait()
        pltpu.sync_copy(rows_v, out_hbm.at[pl.ds(off, CHUNK)])
        return 0

    lax.fori_loop(0, NCHUNK, body, 0)


def kernel(x, table):
    flat = x.reshape(TOTAL).astype(jnp.int32)
    out = _emb_lookup(flat, table)
    return out.reshape(BATCH, HIST, EMBED_DIM)


# speedup vs baseline: 1.1118x; 1.0090x over previous
import functools

import jax
import jax.numpy as jnp
from jax import lax
from jax.experimental import pallas as pl
from jax.experimental.pallas import tpu as pltpu
from jax.experimental.pallas import tpu_sc as plsc

BATCH = 16384
HIST = 50
EMBED_DIM = 32
TOTAL = BATCH * HIST
NUM_CORES = 2
NUM_SUBCORES = 16
NW = NUM_CORES * NUM_SUBCORES
B_PER_W = TOTAL // NW
CHUNK = 1024
NCHUNK = B_PER_W // CHUNK
NBUF = 3

_mesh = plsc.VectorSubcoreMesh(core_axis_name="c", subcore_axis_name="s")


@functools.partial(
    pl.kernel,
    mesh=_mesh,
    out_type=jax.ShapeDtypeStruct((TOTAL, EMBED_DIM), jnp.float32),
    scratch_types=[
        pltpu.VMEM((B_PER_W,), jnp.int32),
        pltpu.VMEM((NBUF, CHUNK, EMBED_DIM), jnp.float32),
        pltpu.SemaphoreType.DMA((NBUF,)),
        pltpu.SemaphoreType.DMA((NBUF,)),
    ],
    compiler_params=pltpu.CompilerParams(use_tc_tiling_on_sc=False),
)
def _emb_lookup(x_hbm, table_hbm, out_hbm, idx_v, rows_v, gsem, wsem):
    wid = lax.axis_index("s") * NUM_CORES + lax.axis_index("c")
    base = wid * B_PER_W

    pltpu.sync_copy(x_hbm.at[pl.ds(base, B_PER_W)], idx_v)

    def stage_gather(j, b):
        return pltpu.async_copy(
            table_hbm.at[idx_v.at[pl.ds(j * CHUNK, CHUNK)]],
            rows_v.at[b],
            gsem.at[b],
        )

    def stage_put(j, b):
        return pltpu.async_copy(
            rows_v.at[b],
            out_hbm.at[pl.ds(base + j * CHUNK, CHUNK)],
            wsem.at[b],
        )

    hg = [None] * NCHUNK
    hw = [None] * NCHUNK
    for j in range(NCHUNK + NBUF - 1):
        if j < NCHUNK:
            b = j % NBUF
            if j >= NBUF:
                hw[j - NBUF].wait()
            hg[j] = stage_gather(j, b)
        k = j - (NBUF - 1)
        if 0 <= k < NCHUNK:
            hg[k].wait()
            hw[k] = stage_put(k, k % NBUF)
    for k in range(max(0, NCHUNK - NBUF), NCHUNK):
        hw[k].wait()


def kernel(x, table):
    flat = x.reshape(TOTAL).astype(jnp.int32)
    out = _emb_lookup(flat, table)
    return out.reshape(BATCH, HIST, EMBED_DIM)


# speedup vs baseline: 1.5189x; 1.3661x over previous
import functools

import jax
import jax.numpy as jnp
from jax import lax
from jax.experimental import pallas as pl
from jax.experimental.pallas import tpu as pltpu
from jax.experimental.pallas import tpu_sc as plsc

BATCH = 16384
HIST = 50
EMBED_DIM = 32
NUM_CORES = 2
NUM_SUBCORES = 16
NW = NUM_CORES * NUM_SUBCORES
BB = 128
NBB = BATCH // BB
BB_PER_W = NBB // NW
NITEM = HIST * BB_PER_W

_mesh = plsc.VectorSubcoreMesh(core_axis_name="c", subcore_axis_name="s")


@functools.partial(
    pl.kernel,
    mesh=_mesh,
    out_type=jax.ShapeDtypeStruct((HIST, 4, NBB, 8, BB), jnp.float32),
    scratch_types=[
        pltpu.VMEM((2, BB), jnp.int32),
        pltpu.VMEM((2, BB, EMBED_DIM), jnp.float32),
        pltpu.VMEM((2, EMBED_DIM, BB), jnp.float32),
        pltpu.SemaphoreType.DMA((2,)),
        pltpu.SemaphoreType.DMA((2,)),
    ],
    compiler_params=pltpu.CompilerParams(
        use_tc_tiling_on_sc=False, needs_layout_passes=False
    ),
)
def _emb_lookup(xt_hbm, table_hbm, out_hbm, idx_v, gbuf, obuf, gsem, wsem):
    wid = lax.axis_index("s") * NUM_CORES + lax.axis_index("c")
    col0 = wid * (BB_PER_W * BB)

    def item_hj(t):
        return t // BB_PER_W, t % BB_PER_W

    def load_idx_and_gather(t, b):
        h, j = item_hj(t)
        pltpu.sync_copy(xt_hbm.at[h, pl.ds(col0 + j * BB, BB)], idx_v.at[b])
        return pltpu.async_copy(table_hbm.at[idx_v.at[b]], gbuf.at[b], gsem.at[b])

    def wait_gather(b):
        pltpu.make_async_copy(table_hbm.at[idx_v.at[b]], gbuf.at[b], gsem.at[b]).wait()

    def issue_writes(t, b):
        h, j = item_hj(t)
        bb = wid * BB_PER_W + j
        for cb in range(4):
            pltpu.async_copy(
                obuf.at[b, pl.ds(cb * 8, 8), :],
                out_hbm.at[h, cb, bb],
                wsem.at[b],
            )

    def wait_writes(t, b):
        h, j = item_hj(t)
        bb = wid * BB_PER_W + j
        for cb in range(4):
            pltpu.make_async_copy(
                obuf.at[b, pl.ds(cb * 8, 8), :],
                out_hbm.at[h, cb, bb],
                wsem.at[b],
            ).wait()

    lanes = lax.iota(jnp.int32, 16)
    rowsets = [k * 16 + lanes for k in range(BB // 16)]

    def transpose_item(b):
        src = gbuf.at[b]
        dst = obuf.at[b]

        def c_body(c, _):
            cols = jnp.full((16,), c, jnp.int32)
            for k in range(BB // 16):
                vals = plsc.load_gather(src, [rowsets[k], cols])
                dst[c, pl.ds(k * 16, 16)] = vals
            return 0

        lax.fori_loop(0, EMBED_DIM, c_body, 0)

    load_idx_and_gather(0, 0)

    def body(t, _):
        b = lax.rem(t, 2)
        nb = lax.rem(t + 1, 2)

        @pl.when(t < NITEM - 1)
        def _():
            load_idx_and_gather(t + 1, nb)

        wait_gather(b)

        @pl.when(t >= 2)
        def _():
            wait_writes(t - 2, b)

        transpose_item(b)
        issue_writes(t, b)
        return 0

    lax.fori_loop(0, NITEM, body, 0)
    wait_writes(NITEM - 2, (NITEM - 2) % 2)
    wait_writes(NITEM - 1, (NITEM - 1) % 2)


def kernel(x, table):
    xt = x.T.astype(jnp.int32)
    out5 = _emb_lookup(xt, table)
    return out5.transpose(2, 4, 0, 1, 3).reshape(BATCH, HIST, EMBED_DIM)


# speedup vs baseline: 1.6501x; 1.0864x over previous
import functools

import jax
import jax.numpy as jnp
from jax import lax
from jax.experimental import pallas as pl
from jax.experimental.pallas import tpu as pltpu
from jax.experimental.pallas import tpu_sc as plsc

BATCH = 16384
HIST = 50
EMBED_DIM = 32
NUM_CORES = 2
NUM_SUBCORES = 16
NW = NUM_CORES * NUM_SUBCORES
BB = 128
NBB = BATCH // BB
BPW = NBB // NW
ROWS = BPW * BB
NITEM = HIST

_mesh = plsc.VectorSubcoreMesh(core_axis_name="c", subcore_axis_name="s")


@functools.partial(
    pl.kernel,
    mesh=_mesh,
    out_type=jax.ShapeDtypeStruct((HIST, 4, NBB, 8, BB), jnp.float32),
    scratch_types=[
        pltpu.VMEM((2, ROWS), jnp.int32),
        pltpu.VMEM((2, ROWS, EMBED_DIM), jnp.float32),
        pltpu.VMEM((2, 4, BPW, 8, BB), jnp.float32),
        pltpu.SemaphoreType.DMA((2,)),
        pltpu.SemaphoreType.DMA((2,)),
        pltpu.SemaphoreType.DMA((2,)),
    ],
    compiler_params=pltpu.CompilerParams(
        use_tc_tiling_on_sc=False, needs_layout_passes=False
    ),
)
def _emb_lookup(xt_hbm, table_hbm, out_hbm, idx_v, gbuf, obuf, isem, gsem, wsem):
    wid = lax.axis_index("s") * NUM_CORES + lax.axis_index("c")
    col0 = wid * ROWS
    bb0 = wid * BPW

    def idx_copy(t, b):
        return pltpu.async_copy(
            xt_hbm.at[t, pl.ds(col0, ROWS)], idx_v.at[b], isem.at[b]
        )

    def gather_copy(b):
        return pltpu.async_copy(table_hbm.at[idx_v.at[b]], gbuf.at[b], gsem.at[b])

    def write_copies(t, b, do_issue):
        for cb in range(4):
            cp = pltpu.make_async_copy(
                obuf.at[b, cb],
                out_hbm.at[t, cb, pl.ds(bb0, BPW)],
                wsem.at[b],
            )
            if do_issue:
                cp.start()
            else:
                cp.wait()

    lanes = lax.iota(jnp.int32, 16)
    rowsets = [[j * BB + k * 16 + lanes for k in range(BB // 16)]
               for j in range(BPW)]

    def transpose_item(b):
        src = gbuf.at[b]
        dst = obuf.at[b]

        def c_body(c, _):
            cb = c // 8
            ci = lax.rem(c, 8)
            cols = jnp.full((16,), c, jnp.int32)
            for j in range(BPW):
                for k in range(BB // 16):
                    vals = plsc.load_gather(src, [rowsets[j][k], cols])
                    dst[cb, j, ci, pl.ds(k * 16, 16)] = vals
            return 0

        lax.fori_loop(0, EMBED_DIM, c_body, 0)

    idx_copy(0, 0).wait()
    gather_copy(0)
    idx_copy(1, 1)

    def body(t, _):
        b = lax.rem(t, 2)
        nb = lax.rem(t + 1, 2)

        pltpu.make_async_copy(
            table_hbm.at[idx_v.at[b]], gbuf.at[b], gsem.at[b]
        ).wait()

        @pl.when(t < NITEM - 2)
        def _():
            idx_copy(t + 2, b)

        @pl.when(t < NITEM - 1)
        def _():
            pltpu.make_async_copy(
                xt_hbm.at[t + 1, pl.ds(col0, ROWS)], idx_v.at[nb], isem.at[nb]
            ).wait()
            gather_copy(nb)

        @pl.when(t >= 2)
        def _():
            write_copies(t - 2, b, do_issue=False)

        transpose_item(b)
        write_copies(t, b, do_issue=True)
        return 0

    lax.fori_loop(0, NITEM, body, 0)
    write_copies(NITEM - 2, (NITEM - 2) % 2, do_issue=False)
    write_copies(NITEM - 1, (NITEM - 1) % 2, do_issue=False)


def kernel(x, table):
    xt = x.T.astype(jnp.int32)
    out5 = _emb_lookup(xt, table)
    return out5.transpose(2, 4, 0, 1, 3).reshape(BATCH, HIST, EMBED_DIM)


# speedup vs baseline: 2.9215x; 1.7704x over previous
import functools

import jax
import jax.numpy as jnp
from jax import lax
from jax.experimental import pallas as pl
from jax.experimental.pallas import tpu as pltpu
from jax.experimental.pallas import tpu_sc as plsc

BATCH = 16384
HIST = 50
EMBED_DIM = 32
NUM_CORES = 2
NUM_SUBCORES = 16
NW = NUM_CORES * NUM_SUBCORES
BB = 128
NBB = BATCH // BB
BPW = NBB // NW
ROWS = BPW * BB
NITEM = HIST

_mesh = plsc.VectorSubcoreMesh(core_axis_name="c", subcore_axis_name="s")


@functools.partial(
    pl.kernel,
    mesh=_mesh,
    out_type=jax.ShapeDtypeStruct((HIST, 4, NBB, 8, BB), jnp.float32),
    scratch_types=[
        pltpu.VMEM((2, ROWS), jnp.int32),
        pltpu.VMEM((2, ROWS, EMBED_DIM), jnp.float32),
        pltpu.VMEM((2, 16, 10, 129), jnp.float32),
        pltpu.SemaphoreType.DMA((2,)),
        pltpu.SemaphoreType.DMA((2,)),
        pltpu.SemaphoreType.DMA((2,)),
    ],
    compiler_params=pltpu.CompilerParams(
        use_tc_tiling_on_sc=False, needs_layout_passes=False
    ),
)
def _emb_lookup(xt_hbm, table_hbm, out_hbm, idx_v, gbuf, obuf, isem, gsem, wsem):
    wid = lax.axis_index("s") * NUM_CORES + lax.axis_index("c")
    col0 = wid * ROWS
    bb0 = wid * BPW

    def idx_copy(t, b):
        return pltpu.async_copy(
            xt_hbm.at[t, pl.ds(col0, ROWS)], idx_v.at[b], isem.at[b]
        )

    def gather_copy(b):
        return pltpu.async_copy(table_hbm.at[idx_v.at[b]], gbuf.at[b], gsem.at[b])

    def write_copies(t, b, do_issue):
        for cb in range(4):
            cp = pltpu.make_async_copy(
                obuf.at[b, pl.ds(cb * BPW, BPW), pl.ds(0, 8), pl.ds(0, BB)],
                out_hbm.at[t, cb, pl.ds(bb0, BPW)],
                wsem.at[b],
            )
            if do_issue:
                cp.start()
            else:
                cp.wait()

    lanes = lax.iota(jnp.int32, 16)
    cb4_lo = (lanes // 8) * BPW
    cb4_hi = ((lanes + 16) // 8) * BPW
    ci_vec = lax.rem(lanes, 8)

    def transpose_item(b):
        src = gbuf.at[b]
        dst = obuf.at[b]

        def blk_body(rb, _):
            for u in range(8):
                r = rb * 8 + u
                j = r // BB
                bi = lax.rem(r, BB)
                ja = jnp.full((16,), j, jnp.int32)
                bia = jnp.full((16,), bi, jnp.int32)
                v_lo = src[r, pl.ds(0, 16)]
                v_hi = src[r, pl.ds(16, 16)]
                plsc.store_scatter(dst, [cb4_lo + ja, ci_vec, bia], v_lo)
                plsc.store_scatter(dst, [cb4_hi + ja, ci_vec, bia], v_hi)
            return 0

        lax.fori_loop(0, ROWS // 8, blk_body, 0)

    idx_copy(0, 0).wait()
    gather_copy(0)
    idx_copy(1, 1)

    def body(t, _):
        b = lax.rem(t, 2)
        nb = lax.rem(t + 1, 2)

        pltpu.make_async_copy(
            table_hbm.at[idx_v.at[b]], gbuf.at[b], gsem.at[b]
        ).wait()

        @pl.when(t < NITEM - 2)
        def _():
            idx_copy(t + 2, b)

        @pl.when(t < NITEM - 1)
        def _():
            pltpu.make_async_copy(
                xt_hbm.at[t + 1, pl.ds(col0, ROWS)], idx_v.at[nb], isem.at[nb]
            ).wait()
            gather_copy(nb)

        @pl.when(t >= 2)
        def _():
            write_copies(t - 2, b, do_issue=False)

        transpose_item(b)
        write_copies(t, b, do_issue=True)
        return 0

    lax.fori_loop(0, NITEM, body, 0)
    write_copies(NITEM - 2, (NITEM - 2) % 2, do_issue=False)
    write_copies(NITEM - 1, (NITEM - 1) % 2, do_issue=False)


def kernel(x, table):
    xt = x.T.astype(jnp.int32)
    out5 = _emb_lookup(xt, table)
    return out5.transpose(2, 4, 0, 1, 3).reshape(BATCH, HIST, EMBED_DIM)
